# baseline (speedup 1.0000x reference)
import functools

import jax
import jax.numpy as jnp
from jax import lax
from jax.experimental import pallas as pl
from jax.experimental.pallas import tpu as pltpu
from jax.experimental.pallas import tpu_sc as plsc

_VOCAB = 100000
_DIM = 32
_BATCH = 1024

_NC = 2
_NS = 16
_NW = _NC * _NS
_BPW = _BATCH // _NW

_sc_mesh = plsc.VectorSubcoreMesh(
    core_axis_name="c", subcore_axis_name="s", num_cores=_NC, num_subcores=_NS
)


@functools.partial(
    pl.kernel,
    mesh=_sc_mesh,
    compiler_params=pltpu.CompilerParams(use_tc_tiling_on_sc=False),
    out_type=jax.ShapeDtypeStruct((_BATCH, _DIM), jnp.float32),
    scratch_types=[
        pltpu.VMEM((_BPW,), jnp.int32),
        pltpu.VMEM((_BPW, _DIM), jnp.float32),
        pltpu.SemaphoreType.DMA,
    ],
)
def _sc_gather(idx_hbm, table_hbm, out_hbm, idx_v, rows_v, sem):
    wid = lax.axis_index("s") * _NC + lax.axis_index("c")
    base = wid * _BPW
    pltpu.sync_copy(idx_hbm.at[pl.ds(base, _BPW)], idx_v)
    pltpu.async_copy(table_hbm.at[idx_v], rows_v, sem).wait()
    pltpu.sync_copy(rows_v, out_hbm.at[pl.ds(base, _BPW)])


_VT = 2048
_NSTEPS = 48
_MAIN = _NSTEPS * _VT
_TAIL = _VOCAB - _MAIN
_NBUF = 4


def _proj_body(emb_ref, w_ref, b_ref, wt_ref, bt_ref, out_hbm, tail_ref, acc, sems):
    i = pl.program_id(0)
    buf = lax.rem(i, _NBUF)

    @pl.when(i >= _NBUF)
    def _wait_prev():
        pltpu.make_async_copy(
            acc.at[buf],
            out_hbm.at[:, pl.ds((i - _NBUF) * _VT, _VT)],
            sems.at[buf],
        ).wait()

    acc[buf] = (
        lax.dot_general(
            emb_ref[...],
            w_ref[...],
            (((1,), (1,)), ((), ())),
            preferred_element_type=jnp.float32,
        )
        + b_ref[0]
    )
    pltpu.make_async_copy(
        acc.at[buf], out_hbm.at[:, pl.ds(i * _VT, _VT)], sems.at[buf]
    ).start()

    @pl.when(i == _NSTEPS - 1)
    def _tail_and_drain():
        tail_ref[...] = (
            lax.dot_general(
                emb_ref[...],
                wt_ref[...],
                (((1,), (1,)), ((), ())),
                preferred_element_type=jnp.float32,
            )
            + bt_ref[...]
        )
        last = _NSTEPS - 1
        for k in range(_NBUF):
            s = last - ((last - k) % _NBUF)
            pltpu.make_async_copy(
                acc.at[k], out_hbm.at[:, pl.ds(s * _VT, _VT)], sems.at[k]
            ).wait()


def _project(emb, W, b3, w_tail, b_tail):
    return pl.pallas_call(
        _proj_body,
        grid=(_NSTEPS,),
        in_specs=[
            pl.BlockSpec((_BATCH, _DIM), lambda i: (0, 0)),
            pl.BlockSpec((_VT, _DIM), lambda i: (i, 0)),
            pl.BlockSpec((1, 1, _VT), lambda i: (i, 0, 0)),
            pl.BlockSpec((_TAIL, _DIM), lambda i: (0, 0)),
            pl.BlockSpec((1, _TAIL), lambda i: (0, 0)),
        ],
        out_specs=[
            pl.BlockSpec(memory_space=pl.ANY),
            pl.BlockSpec((_BATCH, _TAIL), lambda i: (0, 0)),
        ],
        out_shape=[
            jax.ShapeDtypeStruct((_BATCH, _VOCAB), jnp.float32),
            jax.ShapeDtypeStruct((_BATCH, _TAIL), jnp.float32),
        ],
        scratch_shapes=[
            pltpu.VMEM((_NBUF, _BATCH, _VT), jnp.float32),
            pltpu.SemaphoreType.DMA((_NBUF,)),
        ],
    )(emb, W, b3, w_tail, b_tail)


def kernel(prev_tokens, emb_table, W, b):
    emb = _sc_gather(prev_tokens.astype(jnp.int32), emb_table)
    b3 = b[:_MAIN].reshape(_NSTEPS, 1, _VT)
    w_tail = W[_MAIN:]
    b_tail = b[_MAIN:].reshape(1, _TAIL)
    main, tail = _project(emb, W, b3, w_tail, b_tail)
    return lax.dynamic_update_slice(main, tail, (0, _MAIN))


# speedup vs baseline: 1.0008x; 1.0008x over previous
import functools

import jax
import jax.numpy as jnp
from jax import lax
from jax.experimental import pallas as pl
from jax.experimental.pallas import tpu as pltpu
from jax.experimental.pallas import tpu_sc as plsc

_VOCAB = 100000
_DIM = 32
_BATCH = 1024
_GRP = 128 // _DIM
_VG = _VOCAB // _GRP

_NC = 2
_NS = 16
_NW = _NC * _NS
_BPW = _BATCH // _NW

_sc_mesh = plsc.VectorSubcoreMesh(
    core_axis_name="c", subcore_axis_name="s", num_cores=_NC, num_subcores=_NS
)


@functools.partial(
    pl.kernel,
    mesh=_sc_mesh,
    out_type=jax.ShapeDtypeStruct((_BATCH, 128), jnp.float32),
    scratch_types=[
        pltpu.VMEM((_BPW,), jnp.int32),
        pltpu.VMEM((_BPW, 128), jnp.float32),
        pltpu.SemaphoreType.DMA,
    ],
)
def _sc_gather(idx_hbm, table_hbm, out_hbm, idx_v, rows_v, sem):
    wid = lax.axis_index("s") * _NC + lax.axis_index("c")
    base = wid * _BPW
    pltpu.sync_copy(idx_hbm.at[pl.ds(base, _BPW)], idx_v)
    pltpu.async_copy(table_hbm.at[idx_v], rows_v, sem).wait()
    pltpu.sync_copy(rows_v, out_hbm.at[pl.ds(base, _BPW)])


_VT = 2048
_NSTEPS = 48
_MAIN = _NSTEPS * _VT
_TAIL = _VOCAB - _MAIN
_NBUF = 4


def _proj_body(
    emb128_ref, sel_ref, w_ref, b_ref, wt_ref, bt_ref, out_hbm, tail_ref, emb_s, acc, sems
):
    i = pl.program_id(0)
    buf = lax.rem(i, _NBUF)

    @pl.when(i == 0)
    def _select_subrow():
        e128 = emb128_ref[...]
        lo = sel_ref[...]
        e = jnp.zeros((_BATCH, _DIM), jnp.float32)
        for k in range(_GRP):
            e = e + jnp.where(lo == k, e128[:, k * _DIM : (k + 1) * _DIM], 0.0)
        emb_s[...] = e

    @pl.when(i >= _NBUF)
    def _wait_prev():
        pltpu.make_async_copy(
            acc.at[buf],
            out_hbm.at[:, pl.ds((i - _NBUF) * _VT, _VT)],
            sems.at[buf],
        ).wait()

    acc[buf] = (
        lax.dot_general(
            emb_s[...],
            w_ref[...],
            (((1,), (1,)), ((), ())),
            preferred_element_type=jnp.float32,
        )
        + b_ref[0]
    )
    pltpu.make_async_copy(
        acc.at[buf], out_hbm.at[:, pl.ds(i * _VT, _VT)], sems.at[buf]
    ).start()

    @pl.when(i == _NSTEPS - 1)
    def _tail_and_drain():
        tail_ref[...] = (
            lax.dot_general(
                emb_s[...],
                wt_ref[...],
                (((1,), (1,)), ((), ())),
                preferred_element_type=jnp.float32,
            )
            + bt_ref[...]
        )
        last = _NSTEPS - 1
        for k in range(_NBUF):
            s = last - ((last - k) % _NBUF)
            pltpu.make_async_copy(
                acc.at[k], out_hbm.at[:, pl.ds(s * _VT, _VT)], sems.at[k]
            ).wait()


def _project(emb128, sel, W, b3, w_tail, b_tail):
    return pl.pallas_call(
        _proj_body,
        grid=(_NSTEPS,),
        in_specs=[
            pl.BlockSpec((_BATCH, 128), lambda i: (0, 0)),
            pl.BlockSpec((_BATCH, 1), lambda i: (0, 0)),
            pl.BlockSpec((_VT, _DIM), lambda i: (i, 0)),
            pl.BlockSpec((1, 1, _VT), lambda i: (i, 0, 0)),
            pl.BlockSpec((_TAIL, _DIM), lambda i: (0, 0)),
            pl.BlockSpec((1, _TAIL), lambda i: (0, 0)),
        ],
        out_specs=[
            pl.BlockSpec(memory_space=pl.ANY),
            pl.BlockSpec((_BATCH, _TAIL), lambda i: (0, 0)),
        ],
        out_shape=[
            jax.ShapeDtypeStruct((_BATCH, _VOCAB), jnp.float32),
            jax.ShapeDtypeStruct((_BATCH, _TAIL), jnp.float32),
        ],
        scratch_shapes=[
            pltpu.VMEM((_BATCH, _DIM), jnp.float32),
            pltpu.VMEM((_NBUF, _BATCH, _VT), jnp.float32),
            pltpu.SemaphoreType.DMA((_NBUF,)),
        ],
    )(emb128, sel, W, b3, w_tail, b_tail)


def kernel(prev_tokens, emb_table, W, b):
    idx = prev_tokens.astype(jnp.int32)
    table128 = emb_table.reshape(_VG, 128)
    emb128 = _sc_gather(idx // _GRP, table128)
    sel = (idx % _GRP).reshape(_BATCH, 1)
    b3 = b[:_MAIN].reshape(_NSTEPS, 1, _VT)
    w_tail = W[_MAIN:]
    b_tail = b[_MAIN:].reshape(1, _TAIL)
    main, tail = _project(emb128, sel, W, b3, w_tail, b_tail)
    return lax.dynamic_update_slice(main, tail, (0, _MAIN))
